# baseline (speedup 1.0000x reference)
import functools

import jax
import jax.numpy as jnp
from jax import lax
from jax.experimental import pallas as pl
from jax.experimental.pallas import tpu as pltpu
from jax.experimental.pallas import tpu_sc as plsc

RADIUS2 = 0.15 * 0.15
NSAMPLE = 32
B = 8
N = 8192
NPOINT = 1024
FEAT = 64
CH = 3 + FEAT
ROW = NSAMPLE * CH
LANES = 16
NWORKERS = 32
CPW = B * NPOINT // NWORKERS
WPB = NPOINT // CPW
NCHUNK = N // LANES
GCH = 128
NGROUP = NCHUNK // GCH
UNROLL = 8


def _sc_body(xyz_hbm, cent_hbm, feat_hbm, out_hbm,
             xyzi, centv, colbuf, gidx, fbuf, asm, semg):
    cid = lax.axis_index("c")
    sid = lax.axis_index("s")
    wid = cid * 16 + sid
    b = wid // WPB
    q = wid % WPB

    pltpu.sync_copy(xyz_hbm.at[b], xyzi)
    pltpu.sync_copy(cent_hbm.at[b], centv)

    iota = lax.iota(jnp.int32, LANES)
    zeros16 = jnp.zeros((LANES,), jnp.int32)

    def scan_centroid(i):
        ci = q * CPW + i
        cx = plsc.load_gather(centv, [jnp.full((LANES,), ci, jnp.int32)])
        cy = plsc.load_gather(centv, [jnp.full((LANES,), NPOINT + ci, jnp.int32)])
        cz = plsc.load_gather(centv, [jnp.full((LANES,), 2 * NPOINT + ci, jnp.int32)])
        colbuf[0:LANES] = zeros16

        def group(carry):
            g, off0 = carry

            def chunk(c, off):
                base = c * LANES
                px = xyzi[pl.ds(base, LANES)]
                py = xyzi[pl.ds(N + base, LANES)]
                pz = xyzi[pl.ds(2 * N + base, LANES)]
                dx = px - cx
                dy = py - cy
                dz = pz - cz
                d2 = dx * dx + dy * dy + dz * dz
                m = d2 <= RADIUS2
                plsc.store_compressed(colbuf.at[pl.ds(off, LANES)],
                                      c * LANES + iota, mask=m)
                n = plsc.all_reduce_population_count(m)[0]
                return jnp.minimum(off + n, 48)

            off1 = plsc.parallel_loop(g * GCH, (g + 1) * GCH, carry=off0,
                                      unroll=UNROLL)(chunk)
            return g + 1, off1

        _, cnt = lax.while_loop(
            lambda s: jnp.logical_and(s[0] < NGROUP, s[1] < NSAMPLE),
            group, (jnp.int32(0), jnp.int32(0)))
        k = jnp.minimum(cnt, NSAMPLE)
        v0 = colbuf[0:LANES]
        v1 = colbuf[LANES:2 * LANES]
        pad = jnp.full((LANES,), v0[0], jnp.int32)
        sel0 = jnp.where(iota < k, v0, pad)
        sel1 = jnp.where(iota + LANES < k, v1, pad)
        rel = (plsc.load_gather(xyzi, [sel0]) - cx,
               plsc.load_gather(xyzi, [N + sel0]) - cy,
               plsc.load_gather(xyzi, [2 * N + sel0]) - cz,
               plsc.load_gather(xyzi, [sel1]) - cx,
               plsc.load_gather(xyzi, [N + sel1]) - cy,
               plsc.load_gather(xyzi, [2 * N + sel1]) - cz)
        return sel0, sel1, rel

    def stage_a(i, p):
        sel0, sel1, rel = scan_centroid(i)
        gidx[p, 0:LANES] = b * N + sel0
        gidx[p, LANES:2 * LANES] = b * N + sel1
        pltpu.async_copy(feat_hbm.at[gidx.at[p]], fbuf.at[p], semg.at[p])
        a = asm.at[p]
        r0 = iota * CH
        r1 = r0 + LANES * CH
        plsc.store_scatter(a, [r0], rel[0])
        plsc.store_scatter(a, [r0 + 1], rel[1])
        plsc.store_scatter(a, [r0 + 2], rel[2])
        plsc.store_scatter(a, [r1], rel[3])
        plsc.store_scatter(a, [r1 + 1], rel[4])
        plsc.store_scatter(a, [r1 + 2], rel[5])

    def stage_b(i, p):
        pltpu.make_async_copy(feat_hbm.at[gidx.at[p]], fbuf.at[p],
                              semg.at[p]).wait()
        for j in range(NSAMPLE):
            for t in range(FEAT // LANES):
                v = fbuf[p, j, t * LANES:(t + 1) * LANES]
                o = j * CH + 3 + t * LANES
                asm[p, o:o + LANES] = v
        ci = q * CPW + i
        pltpu.sync_copy(asm.at[p], out_hbm.at[b, ci])

    stage_a(0, 0)

    def body(j, carry):
        i0 = 2 * j
        stage_a(i0 + 1, 1)
        stage_b(i0, 0)
        stage_a(i0 + 2, 0)
        stage_b(i0 + 1, 1)
        return carry

    lax.fori_loop(0, (CPW - 2) // 2, body, 0)
    stage_a(CPW - 1, 1)
    stage_b(CPW - 2, 0)
    stage_b(CPW - 1, 1)


_qag = functools.partial(
    pl.kernel,
    out_type=jax.ShapeDtypeStruct((B, NPOINT, ROW), jnp.float32),
    mesh=plsc.VectorSubcoreMesh(core_axis_name="c", subcore_axis_name="s"),
    scratch_types=[
        pltpu.VMEM((3 * N,), jnp.float32),
        pltpu.VMEM((3 * NPOINT,), jnp.float32),
        pltpu.VMEM((64,), jnp.int32),
        pltpu.VMEM((2, NSAMPLE), jnp.int32),
        pltpu.VMEM((2, NSAMPLE, FEAT), jnp.float32),
        pltpu.VMEM((2, ROW), jnp.float32),
        pltpu.SemaphoreType.DMA((2,)),
    ],
    compiler_params=pltpu.CompilerParams(needs_layout_passes=False,
                                         use_tc_tiling_on_sc=False),
)(_sc_body)


@jax.jit
def kernel(xyz, new_xyz, features):
    xyzf = xyz.transpose(0, 2, 1).reshape(B, 3 * N)
    cent = new_xyz.transpose(0, 2, 1).reshape(B, 3 * NPOINT)
    featf = features.reshape(B * N, FEAT)
    out = _qag(xyzf, cent, featf)
    return out.reshape(B, NPOINT, NSAMPLE, CH)


# speedup vs baseline: 1.0337x; 1.0337x over previous
import functools

import jax
import jax.numpy as jnp
from jax import lax
from jax.experimental import pallas as pl
from jax.experimental.pallas import tpu as pltpu
from jax.experimental.pallas import tpu_sc as plsc

RADIUS2 = 0.15 * 0.15
NSAMPLE = 32
B = 8
N = 8192
NPOINT = 1024
FEAT = 64
CH = 3 + FEAT
ROW = NSAMPLE * CH
LANES = 16
NWORKERS = 32
CPW = B * NPOINT // NWORKERS
WPB = NPOINT // CPW
NCHUNK = N // LANES
GCH = 64
NGROUP = NCHUNK // GCH
UNROLL = 8


def _sc_body(xyz_hbm, cent_hbm, feat_hbm, out_hbm,
             xyzi, centv, colbuf, gidx, fbuf, asm, semg):
    cid = lax.axis_index("c")
    sid = lax.axis_index("s")
    wid = cid * 16 + sid
    b = wid // WPB
    q = wid % WPB

    pltpu.sync_copy(xyz_hbm.at[b], xyzi)
    pltpu.sync_copy(cent_hbm.at[b], centv)

    iota = lax.iota(jnp.int32, LANES)
    zeros16 = jnp.zeros((LANES,), jnp.int32)

    def scan_centroid(i):
        ci = q * CPW + i
        cx = plsc.load_gather(centv, [jnp.full((LANES,), ci, jnp.int32)])
        cy = plsc.load_gather(centv, [jnp.full((LANES,), NPOINT + ci, jnp.int32)])
        cz = plsc.load_gather(centv, [jnp.full((LANES,), 2 * NPOINT + ci, jnp.int32)])
        colbuf[0:LANES] = zeros16

        def group(carry):
            g, off0 = carry

            def chunk(c, off):
                base = c * LANES
                px = xyzi[pl.ds(base, LANES)]
                py = xyzi[pl.ds(N + base, LANES)]
                pz = xyzi[pl.ds(2 * N + base, LANES)]
                dx = px - cx
                dy = py - cy
                dz = pz - cz
                d2 = dx * dx + dy * dy + dz * dz
                m = d2 <= RADIUS2
                plsc.store_compressed(colbuf.at[pl.ds(off, LANES)],
                                      c * LANES + iota, mask=m)
                n = plsc.all_reduce_population_count(m)[0]
                return jnp.minimum(off + n, 48)

            off1 = plsc.parallel_loop(g * GCH, (g + 1) * GCH, carry=off0,
                                      unroll=UNROLL)(chunk)
            return g + 1, off1

        _, cnt = lax.while_loop(
            lambda s: jnp.logical_and(s[0] < NGROUP, s[1] < NSAMPLE),
            group, (jnp.int32(0), jnp.int32(0)))
        k = jnp.minimum(cnt, NSAMPLE)
        v0 = colbuf[0:LANES]
        v1 = colbuf[LANES:2 * LANES]
        pad = jnp.full((LANES,), v0[0], jnp.int32)
        sel0 = jnp.where(iota < k, v0, pad)
        sel1 = jnp.where(iota + LANES < k, v1, pad)
        rel = (plsc.load_gather(xyzi, [sel0]) - cx,
               plsc.load_gather(xyzi, [N + sel0]) - cy,
               plsc.load_gather(xyzi, [2 * N + sel0]) - cz,
               plsc.load_gather(xyzi, [sel1]) - cx,
               plsc.load_gather(xyzi, [N + sel1]) - cy,
               plsc.load_gather(xyzi, [2 * N + sel1]) - cz)
        return sel0, sel1, rel

    def stage_a(i, p):
        sel0, sel1, rel = scan_centroid(i)
        gidx[p, 0:LANES] = b * N + sel0
        gidx[p, LANES:2 * LANES] = b * N + sel1
        pltpu.async_copy(feat_hbm.at[gidx.at[p]], fbuf.at[p], semg.at[p])
        a = asm.at[p]
        r0 = iota * CH
        r1 = r0 + LANES * CH
        plsc.store_scatter(a, [r0], rel[0])
        plsc.store_scatter(a, [r0 + 1], rel[1])
        plsc.store_scatter(a, [r0 + 2], rel[2])
        plsc.store_scatter(a, [r1], rel[3])
        plsc.store_scatter(a, [r1 + 1], rel[4])
        plsc.store_scatter(a, [r1 + 2], rel[5])

    def stage_b(i, p):
        pltpu.make_async_copy(feat_hbm.at[gidx.at[p]], fbuf.at[p],
                              semg.at[p]).wait()
        for j in range(NSAMPLE):
            for t in range(FEAT // LANES):
                v = fbuf[p, j, t * LANES:(t + 1) * LANES]
                o = j * CH + 3 + t * LANES
                asm[p, o:o + LANES] = v
        ci = q * CPW + i
        pltpu.sync_copy(asm.at[p], out_hbm.at[b, ci])

    stage_a(0, 0)

    def body(j, carry):
        i0 = 2 * j
        stage_a(i0 + 1, 1)
        stage_b(i0, 0)
        stage_a(i0 + 2, 0)
        stage_b(i0 + 1, 1)
        return carry

    lax.fori_loop(0, (CPW - 2) // 2, body, 0)
    stage_a(CPW - 1, 1)
    stage_b(CPW - 2, 0)
    stage_b(CPW - 1, 1)


_qag = functools.partial(
    pl.kernel,
    out_type=jax.ShapeDtypeStruct((B, NPOINT, ROW), jnp.float32),
    mesh=plsc.VectorSubcoreMesh(core_axis_name="c", subcore_axis_name="s"),
    scratch_types=[
        pltpu.VMEM((3 * N,), jnp.float32),
        pltpu.VMEM((3 * NPOINT,), jnp.float32),
        pltpu.VMEM((64,), jnp.int32),
        pltpu.VMEM((2, NSAMPLE), jnp.int32),
        pltpu.VMEM((2, NSAMPLE, FEAT), jnp.float32),
        pltpu.VMEM((2, ROW), jnp.float32),
        pltpu.SemaphoreType.DMA((2,)),
    ],
    compiler_params=pltpu.CompilerParams(needs_layout_passes=False,
                                         use_tc_tiling_on_sc=False),
)(_sc_body)


@jax.jit
def kernel(xyz, new_xyz, features):
    xyzf = xyz.transpose(0, 2, 1).reshape(B, 3 * N)
    cent = new_xyz.transpose(0, 2, 1).reshape(B, 3 * NPOINT)
    featf = features.reshape(B * N, FEAT)
    out = _qag(xyzf, cent, featf)
    return out.reshape(B, NPOINT, NSAMPLE, CH)
